# baseline (speedup 1.0000x reference)
import numpy as np
import jax
import jax.numpy as jnp
from jax import lax
from jax.experimental import pallas as pl
from jax.experimental.pallas import tpu as pltpu
from jax.experimental.pallas import tpu_sc as plsc

X, Y, DIM = 64, 64, 128
N = X * Y
NUM_ITERS = 100
LEARNING_RADIUS = 0.5
SIGMA = max(X, Y) / 2.0
COGNITIVE, SOCIAL, INERTIA = 0.01, 0.1, 0.001

_D2_VALUES = np.array(
    sorted({dx * dx + dy * dy for dx in range(-63, 64) for dy in range(-63, 64)}),
    dtype=np.int64)
KC = len(_D2_VALUES)
D2_MAX = int(_D2_VALUES[-1])
CRK_LEN = 8064
_CEILRANK = np.searchsorted(_D2_VALUES, np.arange(CRK_LEN), side="left").astype(np.int32)

KPAD = 1664
CH = KPAD // 16
PT = N // 16
HT = N // 32
VCH = KPAD // 128

_VR = np.full((KPAD,), D2_MAX + 1, np.int64)
_VR[:KC] = _D2_VALUES
_VR13 = _VR.reshape(VCH, 128).astype(np.float32)

_GI, _GJ = np.meshgrid(np.arange(X), np.arange(Y), indexing="ij")
_GLX = _GI.reshape(32, 128).astype(np.float32)
_GLY = _GJ.reshape(32, 128).astype(np.float32)
_GLXC = _GI.reshape(N, 1).astype(np.float32)
_GLYC = _GJ.reshape(N, 1).astype(np.float32)


def _prep(itn_ref, iv_ref, p_ref, gl_ref, glx_ref, gly_ref, vr_ref,
          d2_ref, dd_ref, misc_ref, csuf_ref):
    decay = 1.0 - itn_ref[0:1, 0:1] / NUM_ITERS
    lr = LEARNING_RADIUS * decay
    s2 = (SIGMA * decay) ** 2
    diff = iv_ref[:] - p_ref[:] + 1e-6
    dists = jnp.sqrt(jnp.sum(diff * diff, axis=1, keepdims=True))
    dmin = jnp.min(dists)
    iota = lax.broadcasted_iota(jnp.int32, (N, 1), 0)
    bmu = jnp.min(jnp.where(dists <= dmin, iota, N))
    gl_row = gl_ref[pl.ds(bmu, 1), :]
    bx = gl_row[0:1, 0:1]
    by = gl_row[0:1, 1:2]
    dx = glx_ref[:] - bx
    dy = gly_ref[:] - by
    d2 = dx * dx + dy * dy
    nbhd = jnp.exp(-(d2 / s2))
    t = nbhd + lr
    lo = jnp.zeros((32, 128), jnp.int32)
    hi = jnp.full((32, 128), D2_MAX + 1, jnp.int32)
    for _ in range(13):
        mid = (lo + hi) // 2
        pred = jnp.exp(-(mid.astype(jnp.float32) / s2)) <= t
        hi = jnp.where(pred, mid, hi)
        lo = jnp.where(pred, lo, mid + 1)
    d2_ref[...] = d2.astype(jnp.int32)
    dd_ref[...] = lo
    misc_ref[0:1, :] = p_ref[pl.ds(bmu, 1), :]
    misc_ref[1:2, :] = jnp.zeros((1, 128), jnp.float32)
    misc_ref[1:2, 0:1] = bx
    misc_ref[1:2, 1:2] = by
    misc_ref[1:2, 2:3] = lr
    misc_ref[1:2, 3:4] = s2
    d2c_f = (gl_ref[:, 0:1] - bx) ** 2 + (gl_ref[:, 1:2] - by) ** 2
    for kch in range(VCH):
        cmp = (d2c_f >= vr_ref[kch:kch + 1, :]).astype(jnp.float32)
        csuf_ref[kch:kch + 1, :] = jnp.sum(cmp, axis=0, keepdims=True)


_MESH_CACHE = []


def _get_mesh():
    if not _MESH_CACHE:
        _MESH_CACHE.append(plsc.VectorSubcoreMesh(
            core_axis_name="c", subcore_axis_name="s",
            num_cores=2, num_subcores=16))
    return _MESH_CACHE[0]


def _sc_centroid(p_hbm, d2_hbm, dd_hbm, crk_hbm, zrow_hbm, csuf_hbm,
                 out_hbm,
                 crk_v, d2_v, rank_v, p_v, csuf_v, cnt_v,
                 work, tot1, tot_v, dv_v, rrow_v,
                 sums_sh, tot_sh):
    c = lax.axis_index("c")
    s = lax.axis_index("s")

    pltpu.sync_copy(crk_hbm, crk_v)
    pltpu.sync_copy(csuf_hbm, csuf_v)
    pltpu.sync_copy(d2_hbm.at[pl.ds(s * 2, 2), :], d2_v)

    for j in range(2):
        for k in range(8):
            idx = d2_v[j, pl.ds(k * 16, 16)]
            rank_v[j, (k * 16):((k + 1) * 16)] = plsc.load_gather(crk_v, [idx])

    zb = s * CH
    pltpu.sync_copy(zrow_hbm, sums_sh.at[pl.ds(zb, CH), :])
    plsc.subcore_barrier()

    base = s * PT
    for j in range(2):
        idx_row = rank_v.at[j]
        pltpu.sync_copy(p_hbm.at[pl.ds(base + j * 128, 128), :], p_v)
        pltpu.sync_copy(p_v, sums_sh.at[idx_row], add=True)
    plsc.subcore_barrier()

    pltpu.sync_copy(sums_sh.at[pl.ds(zb, CH), :], work)

    def _tot_body(r, acc):
        return tuple(acc[d] + work[r, pl.ds(d * 16, 16)] for d in range(8))

    zero16 = jnp.zeros((16,), jnp.float32)
    tot = lax.fori_loop(0, CH, _tot_body, (zero16,) * 8)
    for d in range(8):
        tot1[0, (d * 16):((d + 1) * 16)] = tot[d]
    pltpu.sync_copy(tot1, tot_sh.at[pl.ds(s, 1), :])
    plsc.subcore_barrier()

    pltpu.sync_copy(tot_sh, tot_v)
    carry = [zero16] * 8
    for k in range(16):
        f = jnp.where(k > s, 1.0, 0.0).astype(jnp.float32)
        for d in range(8):
            carry[d] = carry[d] + f * tot_v[k, pl.ds(d * 16, 16)]

    def _suf_body(i, acc):
        r = CH - 1 - i
        new = tuple(acc[d] + work[r, pl.ds(d * 16, 16)] for d in range(8))
        for d in range(8):
            work[r, pl.ds(d * 16, 16)] = new[d]
        return new

    lax.fori_loop(0, CH, _suf_body, tuple(carry))
    pltpu.sync_copy(work, sums_sh.at[pl.ds(zb, CH), :])
    plsc.subcore_barrier()

    w = c * 16 + s
    gbase = w * HT
    pltpu.sync_copy(dd_hbm.at[pl.ds(w, 1), :], dv_v)
    for k in range(8):
        idx = dv_v[0, pl.ds(k * 16, 16)]
        rk = plsc.load_gather(crk_v, [idx])
        rrow_v[0, (k * 16):((k + 1) * 16)] = rk
        cnt = plsc.load_gather(csuf_v, [rk >> 7, rk & 127])
        cnt_v[pl.ds(k * 16, 16)] = cnt
    gidx = rrow_v.at[0]
    pltpu.sync_copy(sums_sh.at[gidx], p_v)

    def _cen_body(i, _):
        cnt = plsc.load_gather(cnt_v, [jnp.full((16,), i, jnp.int32)])
        for d in range(8):
            p_v[i, pl.ds(d * 16, 16)] = p_v[i, pl.ds(d * 16, 16)] / cnt
        return 0

    lax.fori_loop(0, HT, _cen_body, 0)
    pltpu.sync_copy(p_v, out_hbm.at[pl.ds(gbase, HT), :])


def _update(p_ref, v_ref, r1_ref, r2_ref, cen_ref, glxc_ref, glyc_ref,
            misc_ref, op_ref, ov_ref):
    gbest = misc_ref[0:1, :]
    bx = misc_ref[1:2, 0:1]
    by = misc_ref[1:2, 1:2]
    lr = misc_ref[1:2, 2:3]
    s2 = misc_ref[1:2, 3:4]
    d2c = (glxc_ref[:] - bx) ** 2 + (glyc_ref[:] - by) ** 2
    nbhd = jnp.exp(-(d2c / s2))
    upd = (1.0 - nbhd) <= lr
    p = p_ref[...]
    v = v_ref[...]
    v_upd = (INERTIA * v + COGNITIVE * r1_ref[...] * (cen_ref[...] - p)
             + SOCIAL * r2_ref[...] * (gbest - p))
    ov_ref[...] = jnp.where(upd, v_upd, v)
    op_ref[...] = jnp.where(upd, p + v_upd, p)


def kernel(input_vec, iter_num, particles, velocities, grid_locations, r1, r2):
    itn = jnp.asarray(iter_num, jnp.float32).reshape(1, 1)
    gl_f = jnp.asarray(np.stack([_GI.reshape(-1), _GJ.reshape(-1)], 1)
                       .astype(np.float32))
    iv = input_vec.reshape(1, DIM)

    d2m, ddm, misc, csuf = pl.pallas_call(
        _prep,
        out_shape=[
            jax.ShapeDtypeStruct((32, 128), jnp.int32),
            jax.ShapeDtypeStruct((32, 128), jnp.int32),
            jax.ShapeDtypeStruct((2, 128), jnp.float32),
            jax.ShapeDtypeStruct((VCH, 128), jnp.float32),
        ],
    )(itn, iv, particles, gl_f, jnp.asarray(_GLX), jnp.asarray(_GLY),
      jnp.asarray(_VR13))

    crk = jnp.asarray(_CEILRANK)
    zrow = jnp.zeros((CH, DIM), jnp.float32)

    sc = pl.kernel(
        _sc_centroid,
        out_type=jax.ShapeDtypeStruct((N, DIM), jnp.float32),
        mesh=_get_mesh(),
        compiler_params=pltpu.CompilerParams(needs_layout_passes=False),
        scratch_types=[
            pltpu.VMEM((CRK_LEN,), jnp.int32),
            pltpu.VMEM((2, 128), jnp.int32),
            pltpu.VMEM((2, 128), jnp.int32),
            pltpu.VMEM((128, DIM), jnp.float32),
            pltpu.VMEM((VCH, 128), jnp.float32),
            pltpu.VMEM((HT,), jnp.float32),
            pltpu.VMEM((CH, DIM), jnp.float32),
            pltpu.VMEM((1, DIM), jnp.float32),
            pltpu.VMEM((16, DIM), jnp.float32),
            pltpu.VMEM((1, 128), jnp.int32),
            pltpu.VMEM((1, 128), jnp.int32),
            pltpu.VMEM_SHARED((KPAD, DIM), jnp.float32),
            pltpu.VMEM_SHARED((16, DIM), jnp.float32),
        ],
    )
    cen = sc(particles, d2m, ddm, crk, zrow, csuf)

    BLK = 512
    blk = pl.BlockSpec((BLK, DIM), lambda i: (i, 0))
    cblk = pl.BlockSpec((BLK, 1), lambda i: (i, 0))
    out_p, out_v = pl.pallas_call(
        _update,
        grid=(N // BLK,),
        in_specs=[blk, blk, blk, blk, blk, cblk, cblk,
                  pl.BlockSpec((2, 128), lambda i: (0, 0))],
        out_specs=[blk, blk],
        out_shape=[
            jax.ShapeDtypeStruct((N, DIM), jnp.float32),
            jax.ShapeDtypeStruct((N, DIM), jnp.float32),
        ],
    )(particles, velocities, r1, r2, cen, jnp.asarray(_GLXC), jnp.asarray(_GLYC),
      misc)
    return out_p, out_v


# speedup vs baseline: 1.0471x; 1.0471x over previous
import numpy as np
import jax
import jax.numpy as jnp
from jax import lax
from jax.experimental import pallas as pl
from jax.experimental.pallas import tpu as pltpu
from jax.experimental.pallas import tpu_sc as plsc

X, Y, DIM = 64, 64, 128
N = X * Y
NUM_ITERS = 100
LEARNING_RADIUS = 0.5
SIGMA = max(X, Y) / 2.0
COGNITIVE, SOCIAL, INERTIA = 0.01, 0.1, 0.001

_D2_VALUES = np.array(
    sorted({dx * dx + dy * dy for dx in range(-63, 64) for dy in range(-63, 64)}),
    dtype=np.int64)
KC = len(_D2_VALUES)
D2_MAX = int(_D2_VALUES[-1])
CRK_LEN = 8064
_CEILRANK = np.searchsorted(_D2_VALUES, np.arange(CRK_LEN), side="left").astype(np.int32)

KPAD = 1664
CH = KPAD // 16
PT = N // 16
HT = N // 32
VCH = KPAD // 128

_VR = np.full((KPAD,), D2_MAX + 1, np.int64)
_VR[:KC] = _D2_VALUES
_VR13 = _VR.reshape(VCH, 128).astype(np.int32)

_GI, _GJ = np.meshgrid(np.arange(X), np.arange(Y), indexing="ij")
_GLX = _GI.reshape(32, 128).astype(np.float32)
_GLY = _GJ.reshape(32, 128).astype(np.float32)
_GLXC = _GI.reshape(N, 1).astype(np.float32)
_GLYC = _GJ.reshape(N, 1).astype(np.float32)


def _prep(itn_ref, iv_ref, p_ref, gl_ref, glx_ref, gly_ref, vr_ref,
          d2_ref, dd_ref, misc_ref, csuf_ref):
    decay = 1.0 - itn_ref[0:1, 0:1] / NUM_ITERS
    lr = LEARNING_RADIUS * decay
    s2 = (SIGMA * decay) ** 2
    diff = iv_ref[:] - p_ref[:] + 1e-6
    dists = jnp.sqrt(jnp.sum(diff * diff, axis=1, keepdims=True))
    dmin = jnp.min(dists)
    iota = lax.broadcasted_iota(jnp.int32, (N, 1), 0)
    bmu = jnp.min(jnp.where(dists <= dmin, iota, N))
    gl_row = gl_ref[pl.ds(bmu, 1), :]
    bx = gl_row[0:1, 0:1]
    by = gl_row[0:1, 1:2]
    dx = glx_ref[:] - bx
    dy = gly_ref[:] - by
    d2 = dx * dx + dy * dy
    nbhd = jnp.exp(-(d2 / s2))
    t = nbhd + lr
    lo = jnp.zeros((32, 128), jnp.int32)
    hi = jnp.full((32, 128), D2_MAX + 1, jnp.int32)
    for _ in range(13):
        mid = (lo + hi) // 2
        pred = jnp.exp(-(mid.astype(jnp.float32) / s2)) <= t
        hi = jnp.where(pred, mid, hi)
        lo = jnp.where(pred, lo, mid + 1)
    d2_ref[...] = d2.astype(jnp.int32)
    dd_ref[...] = lo
    misc_ref[0:1, :] = p_ref[pl.ds(bmu, 1), :]
    misc_ref[1:2, :] = jnp.zeros((1, 128), jnp.float32)
    misc_ref[1:2, 0:1] = bx
    misc_ref[1:2, 1:2] = by
    misc_ref[1:2, 2:3] = lr
    misc_ref[1:2, 3:4] = s2
    bi = bx.astype(jnp.int32)
    bj = by.astype(jnp.int32)
    vr = vr_ref[...]
    cnt_lt = jnp.zeros((VCH, 128), jnp.int32)
    for x in range(X):
        q = vr - 1 - (x - bi) * (x - bi)
        qc = jnp.maximum(q, 0)
        w = jnp.sqrt(qc.astype(jnp.float32)).astype(jnp.int32)
        w = jnp.where(w * w > qc, w - 1, w)
        w = jnp.where((w + 1) * (w + 1) <= qc, w + 1, w)
        ln = jnp.minimum(Y - 1, bj + w) - jnp.maximum(0, bj - w) + 1
        cnt_lt = cnt_lt + jnp.where(q < 0, 0, ln)
    csuf_ref[...] = (N - cnt_lt).astype(jnp.float32)


_MESH_CACHE = []


def _get_mesh():
    if not _MESH_CACHE:
        _MESH_CACHE.append(plsc.VectorSubcoreMesh(
            core_axis_name="c", subcore_axis_name="s",
            num_cores=1, num_subcores=16))
    return _MESH_CACHE[0]


def _sc_centroid(p_hbm, d2_hbm, dd_hbm, crk_hbm, zrow_hbm, csuf_hbm,
                 out_hbm,
                 crk_v, d2_v, rank_v, p_v, csuf_v, cnt_v,
                 work, tot1, tot_v, dv_v, rrow_v,
                 sums_sh, tot_sh):
    c = lax.axis_index("c")
    s = lax.axis_index("s")

    pltpu.sync_copy(crk_hbm, crk_v)
    pltpu.sync_copy(csuf_hbm, csuf_v)
    pltpu.sync_copy(d2_hbm.at[pl.ds(s * 2, 2), :], d2_v)

    for j in range(2):
        for k in range(8):
            idx = d2_v[j, pl.ds(k * 16, 16)]
            rank_v[j, (k * 16):((k + 1) * 16)] = plsc.load_gather(crk_v, [idx])

    zb = s * CH
    pltpu.sync_copy(zrow_hbm, sums_sh.at[pl.ds(zb, CH), :])
    plsc.subcore_barrier()

    base = s * PT
    for j in range(2):
        idx_row = rank_v.at[j]
        pltpu.sync_copy(p_hbm.at[pl.ds(base + j * 128, 128), :], p_v)
        pltpu.sync_copy(p_v, sums_sh.at[idx_row], add=True)
    plsc.subcore_barrier()

    pltpu.sync_copy(sums_sh.at[pl.ds(zb, CH), :], work)

    def _tot_body(r, acc):
        return tuple(acc[d] + work[r, pl.ds(d * 16, 16)] for d in range(8))

    zero16 = jnp.zeros((16,), jnp.float32)
    tot = lax.fori_loop(0, CH, _tot_body, (zero16,) * 8)
    for d in range(8):
        tot1[0, (d * 16):((d + 1) * 16)] = tot[d]
    pltpu.sync_copy(tot1, tot_sh.at[pl.ds(s, 1), :])
    plsc.subcore_barrier()

    pltpu.sync_copy(tot_sh, tot_v)
    carry = [zero16] * 8
    for k in range(16):
        f = jnp.where(k > s, 1.0, 0.0).astype(jnp.float32)
        for d in range(8):
            carry[d] = carry[d] + f * tot_v[k, pl.ds(d * 16, 16)]

    def _suf_body(i, acc):
        r = CH - 1 - i
        new = tuple(acc[d] + work[r, pl.ds(d * 16, 16)] for d in range(8))
        for d in range(8):
            work[r, pl.ds(d * 16, 16)] = new[d]
        return new

    lax.fori_loop(0, CH, _suf_body, tuple(carry))
    pltpu.sync_copy(work, sums_sh.at[pl.ds(zb, CH), :])
    plsc.subcore_barrier()

    del c
    for g in range(2):
        w = s * 2 + g
        gbase = w * HT
        pltpu.sync_copy(dd_hbm.at[pl.ds(w, 1), :], dv_v)
        for k in range(8):
            idx = dv_v[0, pl.ds(k * 16, 16)]
            rk = plsc.load_gather(crk_v, [idx])
            rrow_v[0, (k * 16):((k + 1) * 16)] = rk
            cnt = plsc.load_gather(csuf_v, [rk >> 7, rk & 127])
            cnt_v[pl.ds(k * 16, 16)] = cnt
        gidx = rrow_v.at[0]
        pltpu.sync_copy(sums_sh.at[gidx], p_v)

        def _cen_body(i, _):
            cnt = plsc.load_gather(cnt_v, [jnp.full((16,), i, jnp.int32)])
            for d in range(8):
                p_v[i, pl.ds(d * 16, 16)] = p_v[i, pl.ds(d * 16, 16)] / cnt
            return 0

        lax.fori_loop(0, HT, _cen_body, 0)
        pltpu.sync_copy(p_v, out_hbm.at[pl.ds(gbase, HT), :])


def _update(p_ref, v_ref, r1_ref, r2_ref, cen_ref, glxc_ref, glyc_ref,
            misc_ref, op_ref, ov_ref):
    gbest = misc_ref[0:1, :]
    bx = misc_ref[1:2, 0:1]
    by = misc_ref[1:2, 1:2]
    lr = misc_ref[1:2, 2:3]
    s2 = misc_ref[1:2, 3:4]
    d2c = (glxc_ref[:] - bx) ** 2 + (glyc_ref[:] - by) ** 2
    nbhd = jnp.exp(-(d2c / s2))
    upd = (1.0 - nbhd) <= lr
    p = p_ref[...]
    v = v_ref[...]
    v_upd = (INERTIA * v + COGNITIVE * r1_ref[...] * (cen_ref[...] - p)
             + SOCIAL * r2_ref[...] * (gbest - p))
    ov_ref[...] = jnp.where(upd, v_upd, v)
    op_ref[...] = jnp.where(upd, p + v_upd, p)


def kernel(input_vec, iter_num, particles, velocities, grid_locations, r1, r2):
    itn = jnp.asarray(iter_num, jnp.float32).reshape(1, 1)
    gl_f = jnp.asarray(np.stack([_GI.reshape(-1), _GJ.reshape(-1)], 1)
                       .astype(np.float32))
    iv = input_vec.reshape(1, DIM)

    d2m, ddm, misc, csuf = pl.pallas_call(
        _prep,
        out_shape=[
            jax.ShapeDtypeStruct((32, 128), jnp.int32),
            jax.ShapeDtypeStruct((32, 128), jnp.int32),
            jax.ShapeDtypeStruct((2, 128), jnp.float32),
            jax.ShapeDtypeStruct((VCH, 128), jnp.float32),
        ],
    )(itn, iv, particles, gl_f, jnp.asarray(_GLX), jnp.asarray(_GLY),
      jnp.asarray(_VR13))

    crk = jnp.asarray(_CEILRANK)
    zrow = jnp.zeros((CH, DIM), jnp.float32)

    sc = pl.kernel(
        _sc_centroid,
        out_type=jax.ShapeDtypeStruct((N, DIM), jnp.float32),
        mesh=_get_mesh(),
        compiler_params=pltpu.CompilerParams(needs_layout_passes=False),
        scratch_types=[
            pltpu.VMEM((CRK_LEN,), jnp.int32),
            pltpu.VMEM((2, 128), jnp.int32),
            pltpu.VMEM((2, 128), jnp.int32),
            pltpu.VMEM((128, DIM), jnp.float32),
            pltpu.VMEM((VCH, 128), jnp.float32),
            pltpu.VMEM((HT,), jnp.float32),
            pltpu.VMEM((CH, DIM), jnp.float32),
            pltpu.VMEM((1, DIM), jnp.float32),
            pltpu.VMEM((16, DIM), jnp.float32),
            pltpu.VMEM((1, 128), jnp.int32),
            pltpu.VMEM((1, 128), jnp.int32),
            pltpu.VMEM_SHARED((KPAD, DIM), jnp.float32),
            pltpu.VMEM_SHARED((16, DIM), jnp.float32),
        ],
    )
    cen = sc(particles, d2m, ddm, crk, zrow, csuf)

    BLK = 512
    blk = pl.BlockSpec((BLK, DIM), lambda i: (i, 0))
    cblk = pl.BlockSpec((BLK, 1), lambda i: (i, 0))
    out_p, out_v = pl.pallas_call(
        _update,
        grid=(N // BLK,),
        in_specs=[blk, blk, blk, blk, blk, cblk, cblk,
                  pl.BlockSpec((2, 128), lambda i: (0, 0))],
        out_specs=[blk, blk],
        out_shape=[
            jax.ShapeDtypeStruct((N, DIM), jnp.float32),
            jax.ShapeDtypeStruct((N, DIM), jnp.float32),
        ],
    )(particles, velocities, r1, r2, cen, jnp.asarray(_GLXC), jnp.asarray(_GLYC),
      misc)
    return out_p, out_v


# speedup vs baseline: 1.1068x; 1.0570x over previous
import numpy as np
import jax
import jax.numpy as jnp
from jax import lax
from jax.experimental import pallas as pl
from jax.experimental.pallas import tpu as pltpu
from jax.experimental.pallas import tpu_sc as plsc

X, Y, DIM = 64, 64, 128
N = X * Y
NUM_ITERS = 100
LEARNING_RADIUS = 0.5
SIGMA = max(X, Y) / 2.0
COGNITIVE, SOCIAL, INERTIA = 0.01, 0.1, 0.001

_D2_VALUES = np.array(
    sorted({dx * dx + dy * dy for dx in range(-63, 64) for dy in range(-63, 64)}),
    dtype=np.int64)
KC = len(_D2_VALUES)
D2_MAX = int(_D2_VALUES[-1])
CRK_LEN = 8064
_CEILRANK = np.searchsorted(_D2_VALUES, np.arange(CRK_LEN), side="left").astype(np.int32)

KPAD = 1664
CH = KPAD // 16
PT = N // 16
HT = N // 32
VCH = KPAD // 128

_VR = np.full((KPAD,), D2_MAX + 1, np.int64)
_VR[:KC] = _D2_VALUES
_VR13 = _VR.reshape(VCH, 128).astype(np.int32)

_GI, _GJ = np.meshgrid(np.arange(X), np.arange(Y), indexing="ij")
_GLX = _GI.reshape(32, 128).astype(np.float32)
_GLY = _GJ.reshape(32, 128).astype(np.float32)
_GLXC = _GI.reshape(N, 1).astype(np.float32)
_GLYC = _GJ.reshape(N, 1).astype(np.float32)


def _prep(itn_ref, iv_ref, p_ref, gl_ref, glx_ref, gly_ref, vr_ref,
          d2_ref, dd_ref, misc_ref, csuf_ref):
    decay = 1.0 - itn_ref[0:1, 0:1] / NUM_ITERS
    lr = LEARNING_RADIUS * decay
    s2 = (SIGMA * decay) ** 2
    diff = iv_ref[:] - p_ref[:] + 1e-6
    dists = jnp.sqrt(jnp.sum(diff * diff, axis=1, keepdims=True))
    dmin = jnp.min(dists)
    iota = lax.broadcasted_iota(jnp.int32, (N, 1), 0)
    bmu = jnp.min(jnp.where(dists <= dmin, iota, N))
    gl_row = gl_ref[pl.ds(bmu, 1), :]
    bx = gl_row[0:1, 0:1]
    by = gl_row[0:1, 1:2]
    dx = glx_ref[:] - bx
    dy = gly_ref[:] - by
    d2 = dx * dx + dy * dy
    nbhd = jnp.exp(-(d2 / s2))
    t = nbhd + lr
    lo = jnp.zeros((32, 128), jnp.int32)
    hi = jnp.full((32, 128), D2_MAX + 1, jnp.int32)
    for _ in range(13):
        mid = (lo + hi) // 2
        pred = jnp.exp(-(mid.astype(jnp.float32) / s2)) <= t
        hi = jnp.where(pred, mid, hi)
        lo = jnp.where(pred, lo, mid + 1)
    d2_ref[...] = d2.astype(jnp.int32)
    dd_ref[...] = lo
    misc_ref[0:1, :] = p_ref[pl.ds(bmu, 1), :]
    misc_ref[1:2, :] = jnp.zeros((1, 128), jnp.float32)
    misc_ref[1:2, 0:1] = bx
    misc_ref[1:2, 1:2] = by
    misc_ref[1:2, 2:3] = lr
    misc_ref[1:2, 3:4] = s2
    bi = bx.astype(jnp.int32)
    bj = by.astype(jnp.int32)
    vr = vr_ref[...]
    cnt_lt = jnp.zeros((VCH, 128), jnp.int32)
    for x in range(X):
        q = vr - 1 - (x - bi) * (x - bi)
        qc = jnp.maximum(q, 0)
        w = jnp.sqrt(qc.astype(jnp.float32)).astype(jnp.int32)
        w = jnp.where(w * w > qc, w - 1, w)
        w = jnp.where((w + 1) * (w + 1) <= qc, w + 1, w)
        ln = jnp.minimum(Y - 1, bj + w) - jnp.maximum(0, bj - w) + 1
        cnt_lt = cnt_lt + jnp.where(q < 0, 0, ln)
    csuf_ref[...] = (N - cnt_lt).astype(jnp.float32)


_MESH_CACHE = []


def _get_mesh():
    if not _MESH_CACHE:
        _MESH_CACHE.append(plsc.VectorSubcoreMesh(
            core_axis_name="c", subcore_axis_name="s",
            num_cores=1, num_subcores=16))
    return _MESH_CACHE[0]


def _sc_centroid(p_hbm, d2_hbm, dd_hbm, crk_hbm, zrow_hbm, csuf_hbm,
                 out_hbm,
                 crk_v, d2_v, rank_v, p_v, csuf_v,
                 work, tot1, tot_v, dv_v, rrow_v,
                 sums_sh, tot_sh):
    c = lax.axis_index("c")
    s = lax.axis_index("s")

    pltpu.sync_copy(crk_hbm, crk_v)
    pltpu.sync_copy(csuf_hbm, csuf_v)
    pltpu.sync_copy(d2_hbm.at[pl.ds(s * 2, 2), :], d2_v)

    for j in range(2):
        for k in range(8):
            idx = d2_v[j, pl.ds(k * 16, 16)]
            rank_v[j, (k * 16):((k + 1) * 16)] = plsc.load_gather(crk_v, [idx])

    zb = s * CH
    pltpu.sync_copy(zrow_hbm, sums_sh.at[pl.ds(zb, CH), :])
    plsc.subcore_barrier()

    base = s * PT
    for j in range(2):
        idx_row = rank_v.at[j]
        pltpu.sync_copy(p_hbm.at[pl.ds(base + j * 128, 128), :], p_v)
        pltpu.sync_copy(p_v, sums_sh.at[idx_row], add=True)
    plsc.subcore_barrier()

    pltpu.sync_copy(sums_sh.at[pl.ds(zb, CH), :], work)

    def _tot_body(r, acc):
        return tuple(acc[d] + work[r, pl.ds(d * 16, 16)] for d in range(8))

    zero16 = jnp.zeros((16,), jnp.float32)
    tot = lax.fori_loop(0, CH, _tot_body, (zero16,) * 8)
    for d in range(8):
        tot1[0, (d * 16):((d + 1) * 16)] = tot[d]
    pltpu.sync_copy(tot1, tot_sh.at[pl.ds(s, 1), :])
    plsc.subcore_barrier()

    pltpu.sync_copy(tot_sh, tot_v)
    carry = [zero16] * 8
    for k in range(16):
        f = jnp.where(k > s, 1.0, 0.0).astype(jnp.float32)
        for d in range(8):
            carry[d] = carry[d] + f * tot_v[k, pl.ds(d * 16, 16)]

    def _suf_body(i, acc):
        r = CH - 1 - i
        new = tuple(acc[d] + work[r, pl.ds(d * 16, 16)] for d in range(8))
        rr = zb + r
        cnt = plsc.load_gather(
            csuf_v, [jnp.full((16,), rr >> 7, jnp.int32),
                     jnp.full((16,), rr & 127, jnp.int32)])
        for d in range(8):
            work[r, pl.ds(d * 16, 16)] = new[d] / cnt
        return new

    lax.fori_loop(0, CH, _suf_body, tuple(carry))
    pltpu.sync_copy(work, sums_sh.at[pl.ds(zb, CH), :])
    plsc.subcore_barrier()

    del c
    for g in range(2):
        w = s * 2 + g
        gbase = w * HT
        pltpu.sync_copy(dd_hbm.at[pl.ds(w, 1), :], dv_v)
        for k in range(8):
            idx = dv_v[0, pl.ds(k * 16, 16)]
            rrow_v[0, (k * 16):((k + 1) * 16)] = plsc.load_gather(crk_v, [idx])
        gidx = rrow_v.at[0]
        pltpu.sync_copy(sums_sh.at[gidx], p_v)
        pltpu.sync_copy(p_v, out_hbm.at[pl.ds(gbase, HT), :])


def _update(p_ref, v_ref, r1_ref, r2_ref, cen_ref, glxc_ref, glyc_ref,
            misc_ref, op_ref, ov_ref):
    gbest = misc_ref[0:1, :]
    bx = misc_ref[1:2, 0:1]
    by = misc_ref[1:2, 1:2]
    lr = misc_ref[1:2, 2:3]
    s2 = misc_ref[1:2, 3:4]
    d2c = (glxc_ref[:] - bx) ** 2 + (glyc_ref[:] - by) ** 2
    nbhd = jnp.exp(-(d2c / s2))
    upd = (1.0 - nbhd) <= lr
    p = p_ref[...]
    v = v_ref[...]
    v_upd = (INERTIA * v + COGNITIVE * r1_ref[...] * (cen_ref[...] - p)
             + SOCIAL * r2_ref[...] * (gbest - p))
    ov_ref[...] = jnp.where(upd, v_upd, v)
    op_ref[...] = jnp.where(upd, p + v_upd, p)


def kernel(input_vec, iter_num, particles, velocities, grid_locations, r1, r2):
    itn = jnp.asarray(iter_num, jnp.float32).reshape(1, 1)
    gl_f = jnp.asarray(np.stack([_GI.reshape(-1), _GJ.reshape(-1)], 1)
                       .astype(np.float32))
    iv = input_vec.reshape(1, DIM)

    d2m, ddm, misc, csuf = pl.pallas_call(
        _prep,
        out_shape=[
            jax.ShapeDtypeStruct((32, 128), jnp.int32),
            jax.ShapeDtypeStruct((32, 128), jnp.int32),
            jax.ShapeDtypeStruct((2, 128), jnp.float32),
            jax.ShapeDtypeStruct((VCH, 128), jnp.float32),
        ],
    )(itn, iv, particles, gl_f, jnp.asarray(_GLX), jnp.asarray(_GLY),
      jnp.asarray(_VR13))

    crk = jnp.asarray(_CEILRANK)
    zrow = jnp.zeros((CH, DIM), jnp.float32)

    sc = pl.kernel(
        _sc_centroid,
        out_type=jax.ShapeDtypeStruct((N, DIM), jnp.float32),
        mesh=_get_mesh(),
        compiler_params=pltpu.CompilerParams(needs_layout_passes=False),
        scratch_types=[
            pltpu.VMEM((CRK_LEN,), jnp.int32),
            pltpu.VMEM((2, 128), jnp.int32),
            pltpu.VMEM((2, 128), jnp.int32),
            pltpu.VMEM((128, DIM), jnp.float32),
            pltpu.VMEM((VCH, 128), jnp.float32),
            pltpu.VMEM((CH, DIM), jnp.float32),
            pltpu.VMEM((1, DIM), jnp.float32),
            pltpu.VMEM((16, DIM), jnp.float32),
            pltpu.VMEM((1, 128), jnp.int32),
            pltpu.VMEM((1, 128), jnp.int32),
            pltpu.VMEM_SHARED((KPAD, DIM), jnp.float32),
            pltpu.VMEM_SHARED((16, DIM), jnp.float32),
        ],
    )
    cen = sc(particles, d2m, ddm, crk, zrow, csuf)

    BLK = 512
    blk = pl.BlockSpec((BLK, DIM), lambda i: (i, 0))
    cblk = pl.BlockSpec((BLK, 1), lambda i: (i, 0))
    out_p, out_v = pl.pallas_call(
        _update,
        grid=(N // BLK,),
        in_specs=[blk, blk, blk, blk, blk, cblk, cblk,
                  pl.BlockSpec((2, 128), lambda i: (0, 0))],
        out_specs=[blk, blk],
        out_shape=[
            jax.ShapeDtypeStruct((N, DIM), jnp.float32),
            jax.ShapeDtypeStruct((N, DIM), jnp.float32),
        ],
    )(particles, velocities, r1, r2, cen, jnp.asarray(_GLXC), jnp.asarray(_GLYC),
      misc)
    return out_p, out_v


# speedup vs baseline: 1.2192x; 1.1016x over previous
import numpy as np
import jax
import jax.numpy as jnp
from jax import lax
from jax.experimental import pallas as pl
from jax.experimental.pallas import tpu as pltpu
from jax.experimental.pallas import tpu_sc as plsc

X, Y, DIM = 64, 64, 128
N = X * Y
NUM_ITERS = 100
LEARNING_RADIUS = 0.5
SIGMA = max(X, Y) / 2.0
COGNITIVE, SOCIAL, INERTIA = 0.01, 0.1, 0.001

_D2_VALUES = np.array(
    sorted({dx * dx + dy * dy for dx in range(-63, 64) for dy in range(-63, 64)}),
    dtype=np.int64)
KC = len(_D2_VALUES)
D2_MAX = int(_D2_VALUES[-1])
CRK_LEN = 8064
_CEILRANK = np.searchsorted(_D2_VALUES, np.arange(CRK_LEN), side="left").astype(np.int32)

KPAD = 1664
CH = KPAD // 16
PT = N // 16
HT = N // 32
VCH = KPAD // 128

_VR = np.full((KPAD,), D2_MAX + 1, np.int64)
_VR[:KC] = _D2_VALUES
_VR13 = _VR.reshape(VCH, 128).astype(np.int32)

_GI, _GJ = np.meshgrid(np.arange(X), np.arange(Y), indexing="ij")
_GLX = _GI.reshape(32, 128).astype(np.float32)
_GLY = _GJ.reshape(32, 128).astype(np.float32)
_GLXC = _GI.reshape(N, 1).astype(np.float32)
_GLYC = _GJ.reshape(N, 1).astype(np.float32)


def _prep(itn_ref, iv_ref, p_ref, gl_ref, glx_ref, gly_ref, vr_ref,
          d2_ref, dd_ref, misc_ref, csuf_ref):
    decay = 1.0 - itn_ref[0:1, 0:1] / NUM_ITERS
    lr = LEARNING_RADIUS * decay
    s2 = (SIGMA * decay) ** 2
    diff = iv_ref[:] - p_ref[:] + 1e-6
    dists = jnp.sqrt(jnp.sum(diff * diff, axis=1, keepdims=True))
    dmin = jnp.min(dists)
    iota = lax.broadcasted_iota(jnp.int32, (N, 1), 0)
    bmu = jnp.min(jnp.where(dists <= dmin, iota, N))
    gl_row = gl_ref[pl.ds(bmu, 1), :]
    bx = gl_row[0:1, 0:1]
    by = gl_row[0:1, 1:2]
    dx = glx_ref[:] - bx
    dy = gly_ref[:] - by
    d2 = dx * dx + dy * dy
    nbhd = jnp.exp(-(d2 / s2))
    t = nbhd + lr
    lo = jnp.zeros((32, 128), jnp.int32)
    hi = jnp.full((32, 128), D2_MAX + 1, jnp.int32)
    for _ in range(13):
        mid = (lo + hi) // 2
        pred = jnp.exp(-(mid.astype(jnp.float32) / s2)) <= t
        hi = jnp.where(pred, mid, hi)
        lo = jnp.where(pred, lo, mid + 1)
    d2_ref[...] = d2.astype(jnp.int32)
    dd_ref[...] = lo
    misc_ref[0:1, :] = p_ref[pl.ds(bmu, 1), :]
    misc_ref[1:2, :] = jnp.zeros((1, 128), jnp.float32)
    misc_ref[1:2, 0:1] = bx
    misc_ref[1:2, 1:2] = by
    misc_ref[1:2, 2:3] = lr
    misc_ref[1:2, 3:4] = s2
    bi = bx.astype(jnp.int32)
    bj = by.astype(jnp.int32)
    vr = vr_ref[...]
    cnt_lt = jnp.zeros((VCH, 128), jnp.int32)
    for x in range(X):
        q = vr - 1 - (x - bi) * (x - bi)
        qc = jnp.maximum(q, 0)
        w = jnp.sqrt(qc.astype(jnp.float32)).astype(jnp.int32)
        w = jnp.where(w * w > qc, w - 1, w)
        w = jnp.where((w + 1) * (w + 1) <= qc, w + 1, w)
        ln = jnp.minimum(Y - 1, bj + w) - jnp.maximum(0, bj - w) + 1
        cnt_lt = cnt_lt + jnp.where(q < 0, 0, ln)
    csuf_ref[...] = (N - cnt_lt).astype(jnp.float32)


_MESH_CACHE = []


def _get_mesh():
    if not _MESH_CACHE:
        _MESH_CACHE.append(plsc.VectorSubcoreMesh(
            core_axis_name="c", subcore_axis_name="s",
            num_cores=1, num_subcores=16))
    return _MESH_CACHE[0]


def _sc_centroid(p_hbm, d2_hbm, dd_hbm, crk_hbm, zrow_hbm, csuf_hbm,
                 out_hbm,
                 crk_v, d2_v, rank_v, p_v, p_v2, csuf_v,
                 work, tot1, tot_v, dv_v, rrow_v,
                 sums_sh, tot_sh, sem_a, sem_b):
    c = lax.axis_index("c")
    s = lax.axis_index("s")
    zb = s * CH
    base = s * PT

    h_crk = pltpu.async_copy(crk_hbm, crk_v, sem_a)
    h_csuf = pltpu.async_copy(csuf_hbm, csuf_v, sem_a)
    h_d2 = pltpu.async_copy(d2_hbm.at[pl.ds(s * 2, 2), :], d2_v, sem_a)
    h_z = pltpu.async_copy(zrow_hbm, sums_sh.at[pl.ds(zb, CH), :], sem_a)
    h_p0 = pltpu.async_copy(p_hbm.at[pl.ds(base, 128), :], p_v, sem_b)
    h_p1 = pltpu.async_copy(p_hbm.at[pl.ds(base + 128, 128), :], p_v2, sem_b)
    h_crk.wait()
    h_csuf.wait()
    h_d2.wait()

    for j in range(2):
        for k in range(8):
            idx = d2_v[j, pl.ds(k * 16, 16)]
            rank_v[j, (k * 16):((k + 1) * 16)] = plsc.load_gather(crk_v, [idx])

    h_z.wait()
    plsc.subcore_barrier()

    h_p0.wait()
    h_p1.wait()
    pltpu.sync_copy(p_v, sums_sh.at[rank_v.at[0]], add=True)
    pltpu.sync_copy(p_v2, sums_sh.at[rank_v.at[1]], add=True)
    plsc.subcore_barrier()

    pltpu.sync_copy(sums_sh.at[pl.ds(zb, CH), :], work)

    def _tot_body(r, acc):
        return tuple(acc[d] + work[r, pl.ds(d * 16, 16)] for d in range(8))

    zero16 = jnp.zeros((16,), jnp.float32)
    tot = lax.fori_loop(0, CH, _tot_body, (zero16,) * 8)
    for d in range(8):
        tot1[0, (d * 16):((d + 1) * 16)] = tot[d]
    pltpu.sync_copy(tot1, tot_sh.at[pl.ds(s, 1), :])
    plsc.subcore_barrier()

    pltpu.sync_copy(tot_sh, tot_v)
    carry = [zero16] * 8
    for k in range(16):
        f = jnp.where(k > s, 1.0, 0.0).astype(jnp.float32)
        for d in range(8):
            carry[d] = carry[d] + f * tot_v[k, pl.ds(d * 16, 16)]

    def _suf_body(i, acc):
        r = CH - 1 - i
        new = tuple(acc[d] + work[r, pl.ds(d * 16, 16)] for d in range(8))
        rr = zb + r
        cnt = plsc.load_gather(
            csuf_v, [jnp.full((16,), rr >> 7, jnp.int32),
                     jnp.full((16,), rr & 127, jnp.int32)])
        for d in range(8):
            work[r, pl.ds(d * 16, 16)] = new[d] / cnt
        return new

    lax.fori_loop(0, CH, _suf_body, tuple(carry))
    pltpu.sync_copy(work, sums_sh.at[pl.ds(zb, CH), :])
    plsc.subcore_barrier()

    del c
    pltpu.sync_copy(dd_hbm.at[pl.ds(s * 2, 2), :], dv_v)
    for g in range(2):
        for k in range(8):
            idx = dv_v[g, pl.ds(k * 16, 16)]
            rrow_v[g, (k * 16):((k + 1) * 16)] = plsc.load_gather(crk_v, [idx])
    pltpu.sync_copy(sums_sh.at[rrow_v.at[0]], p_v)
    h_o0 = pltpu.async_copy(p_v, out_hbm.at[pl.ds(s * PT, HT), :], sem_a)
    pltpu.sync_copy(sums_sh.at[rrow_v.at[1]], p_v2)
    h_o0.wait()
    h_o1 = pltpu.async_copy(p_v2, out_hbm.at[pl.ds(s * PT + HT, HT), :], sem_a)
    h_o1.wait()


def _update(p_ref, v_ref, r1_ref, r2_ref, cen_ref, glxc_ref, glyc_ref,
            misc_ref, op_ref, ov_ref):
    gbest = misc_ref[0:1, :]
    bx = misc_ref[1:2, 0:1]
    by = misc_ref[1:2, 1:2]
    lr = misc_ref[1:2, 2:3]
    s2 = misc_ref[1:2, 3:4]
    d2c = (glxc_ref[:] - bx) ** 2 + (glyc_ref[:] - by) ** 2
    nbhd = jnp.exp(-(d2c / s2))
    upd = (1.0 - nbhd) <= lr
    p = p_ref[...]
    v = v_ref[...]
    v_upd = (INERTIA * v + COGNITIVE * r1_ref[...] * (cen_ref[...] - p)
             + SOCIAL * r2_ref[...] * (gbest - p))
    ov_ref[...] = jnp.where(upd, v_upd, v)
    op_ref[...] = jnp.where(upd, p + v_upd, p)


def kernel(input_vec, iter_num, particles, velocities, grid_locations, r1, r2):
    itn = jnp.asarray(iter_num, jnp.float32).reshape(1, 1)
    gl_f = jnp.asarray(np.stack([_GI.reshape(-1), _GJ.reshape(-1)], 1)
                       .astype(np.float32))
    iv = input_vec.reshape(1, DIM)

    d2m, ddm, misc, csuf = pl.pallas_call(
        _prep,
        out_shape=[
            jax.ShapeDtypeStruct((32, 128), jnp.int32),
            jax.ShapeDtypeStruct((32, 128), jnp.int32),
            jax.ShapeDtypeStruct((2, 128), jnp.float32),
            jax.ShapeDtypeStruct((VCH, 128), jnp.float32),
        ],
    )(itn, iv, particles, gl_f, jnp.asarray(_GLX), jnp.asarray(_GLY),
      jnp.asarray(_VR13))

    crk = jnp.asarray(_CEILRANK)
    zrow = jnp.zeros((CH, DIM), jnp.float32)

    sc = pl.kernel(
        _sc_centroid,
        out_type=jax.ShapeDtypeStruct((N, DIM), jnp.float32),
        mesh=_get_mesh(),
        compiler_params=pltpu.CompilerParams(needs_layout_passes=False),
        scratch_types=[
            pltpu.VMEM((CRK_LEN,), jnp.int32),
            pltpu.VMEM((2, 128), jnp.int32),
            pltpu.VMEM((2, 128), jnp.int32),
            pltpu.VMEM((128, DIM), jnp.float32),
            pltpu.VMEM((128, DIM), jnp.float32),
            pltpu.VMEM((VCH, 128), jnp.float32),
            pltpu.VMEM((CH, DIM), jnp.float32),
            pltpu.VMEM((1, DIM), jnp.float32),
            pltpu.VMEM((16, DIM), jnp.float32),
            pltpu.VMEM((2, 128), jnp.int32),
            pltpu.VMEM((2, 128), jnp.int32),
            pltpu.VMEM_SHARED((KPAD, DIM), jnp.float32),
            pltpu.VMEM_SHARED((16, DIM), jnp.float32),
            pltpu.SemaphoreType.DMA,
            pltpu.SemaphoreType.DMA,
        ],
    )
    cen = sc(particles, d2m, ddm, crk, zrow, csuf)

    BLK = 512
    blk = pl.BlockSpec((BLK, DIM), lambda i: (i, 0))
    cblk = pl.BlockSpec((BLK, 1), lambda i: (i, 0))
    out_p, out_v = pl.pallas_call(
        _update,
        grid=(N // BLK,),
        in_specs=[blk, blk, blk, blk, blk, cblk, cblk,
                  pl.BlockSpec((2, 128), lambda i: (0, 0))],
        out_specs=[blk, blk],
        out_shape=[
            jax.ShapeDtypeStruct((N, DIM), jnp.float32),
            jax.ShapeDtypeStruct((N, DIM), jnp.float32),
        ],
    )(particles, velocities, r1, r2, cen, jnp.asarray(_GLXC), jnp.asarray(_GLYC),
      misc)
    return out_p, out_v
